# Initial kernel scaffold; baseline (speedup 1.0000x reference)
import jax
import jax.numpy as jnp
from jax.experimental import pallas as pl


def kernel(x, edge_index_0, edge_index_1, proj_W, proj_b, gat1_W_0, gat1_al_0, gat1_ar_0, gat1_b_0, gat2_W_0, gat2_al_0, gat2_ar_0, gat2_b_0, gat1_W_1, gat1_al_1, gat1_ar_1, gat1_b_1, gat2_W_1, gat2_al_1, gat2_ar_1, gat2_b_1, attn_w, cls_W, cls_b):
    raise NotImplementedError("write your pallas kernel here")


# speedup vs baseline: 1.0005x; 1.0005x over previous
import functools

import jax
import jax.numpy as jnp
from jax.experimental import pallas as pl
from jax.experimental.pallas import tpu as pltpu

N = 10000
E = 160000
IN_DIM = 256
HID = 512
HEADS = 8
HPH = 64
OUT = 64


def _elu(v):
    return jnp.where(v > 0, v, jnp.exp(jnp.minimum(v, 0.0)) - 1.0)


def _proj_body(x_ref, w_ref, b_ref, o_ref):
    acc = jnp.dot(x_ref[...], w_ref[...], preferred_element_type=jnp.float32)
    o_ref[...] = _elu(acc + b_ref[...])


def _proj(x, w, b):
    blk = 1000
    return pl.pallas_call(
        _proj_body,
        grid=(N // blk,),
        in_specs=[
            pl.BlockSpec((blk, IN_DIM), lambda i: (i, 0)),
            pl.BlockSpec((IN_DIM, HID), lambda i: (0, 0)),
            pl.BlockSpec((1, HID), lambda i: (0, 0)),
        ],
        out_specs=pl.BlockSpec((blk, HID), lambda i: (i, 0)),
        out_shape=jax.ShapeDtypeStruct((N, HID), jnp.float32),
    )(x, w, b.reshape(1, HID))


def _gat(x, src, dst, W, al, ar, b, heads, oph):
    feat = (x @ W).reshape(N, heads, oph)
    el = (feat * al[None, :, :]).sum(-1)
    er = (feat * ar[None, :, :]).sum(-1)
    e = jax.nn.leaky_relu(el[src] + er[dst], 0.2)
    emax = jax.ops.segment_max(e, dst, num_segments=N)
    emax = jnp.where(jnp.isfinite(emax), emax, 0.0)
    ee = jnp.exp(e - emax[dst])
    den = jax.ops.segment_sum(ee, dst, num_segments=N)
    alpha = ee / (den[dst] + 1e-9)
    msg = feat[src] * alpha[:, :, None]
    rst = jax.ops.segment_sum(msg, dst, num_segments=N)
    return rst + b.reshape(1, heads, oph)


def kernel(x, edge_index_0, edge_index_1, proj_W, proj_b,
           gat1_W_0, gat1_al_0, gat1_ar_0, gat1_b_0,
           gat2_W_0, gat2_al_0, gat2_ar_0, gat2_b_0,
           gat1_W_1, gat1_al_1, gat1_ar_1, gat1_b_1,
           gat2_W_1, gat2_al_1, gat2_ar_1, gat2_b_1,
           attn_w, cls_W, cls_b):
    h_proj = _proj(x, proj_W, proj_b)
    blocks = [
        (gat1_W_0, gat1_al_0, gat1_ar_0, gat1_b_0, gat2_W_0, gat2_al_0, gat2_ar_0, gat2_b_0),
        (gat1_W_1, gat1_al_1, gat1_ar_1, gat1_b_1, gat2_W_1, gat2_al_1, gat2_ar_1, gat2_b_1),
    ]
    edges = [edge_index_0, edge_index_1]
    embs = []
    for (g1W, g1al, g1ar, g1b, g2W, g2al, g2ar, g2b), ei in zip(blocks, edges):
        src = ei[0]
        dst = ei[1]
        h = _gat(h_proj, src, dst, g1W, g1al, g1ar, g1b, HEADS, HPH)
        h = jax.nn.elu(h.reshape(N, HEADS * HPH))
        h = _gat(h, src, dst, g2W, g2al, g2ar, g2b, 1, HID)[:, 0, :]
        embs.append(h)
    H = jnp.stack(embs, axis=1)
    scores = jnp.squeeze(H @ attn_w, -1)
    alpha = jax.nn.softmax(scores, axis=1)
    h = (alpha[:, :, None] * H).sum(axis=1)
    return h @ cls_W + cls_b


# speedup vs baseline: 11.4917x; 11.4858x over previous
import functools

import jax
import jax.numpy as jnp
from jax import lax
from jax.experimental import pallas as pl
from jax.experimental.pallas import tpu as pltpu
from jax.experimental.pallas import tpu_sc as plsc

N = 10000
E = 160000
IN_DIM = 256
HID = 512
HEADS = 8
HPH = 64
OUT = 64

BLK = 1000
NEG = -1e30


def _elu(v):
    return jnp.where(v > 0, v, jnp.exp(jnp.minimum(v, 0.0)) - 1.0)


def _proj_body(x_ref, w_ref, b_ref, o_ref):
    acc = jnp.dot(x_ref[...], w_ref[...], preferred_element_type=jnp.float32)
    o_ref[...] = _elu(acc + b_ref[...])


def _proj(x, w, b):
    return pl.pallas_call(
        _proj_body,
        grid=(N // BLK,),
        in_specs=[
            pl.BlockSpec((BLK, IN_DIM), lambda i: (i, 0)),
            pl.BlockSpec((IN_DIM, HID), lambda i: (0, 0)),
            pl.BlockSpec((1, HID), lambda i: (0, 0)),
        ],
        out_specs=pl.BlockSpec((BLK, HID), lambda i: (i, 0)),
        out_shape=jax.ShapeDtypeStruct((N, HID), jnp.float32),
    )(x, w, b.reshape(1, HID))


def _featlogit_body(h_ref, w_ref, alm_ref, arm_ref,
                    feat_ref, el_ref, er_ref, m_ref, mel_acc, mer_acc):
    i = pl.program_id(0)
    f = jnp.dot(h_ref[...], w_ref[...], preferred_element_type=jnp.float32)
    feat_ref[...] = f
    el = jnp.dot(f, alm_ref[...], preferred_element_type=jnp.float32)
    er = jnp.dot(f, arm_ref[...], preferred_element_type=jnp.float32)
    el_ref[...] = el
    er_ref[...] = er

    @pl.when(i == 0)
    def _():
        mel_acc[...] = jnp.full((8, 128), NEG, jnp.float32)
        mer_acc[...] = jnp.full((8, 128), NEG, jnp.float32)

    mel_acc[...] = jnp.maximum(mel_acc[...], jnp.full((8, 128), jnp.max(el)))
    mer_acc[...] = jnp.maximum(mer_acc[...], jnp.full((8, 128), jnp.max(er)))
    m_ref[...] = (mel_acc[...] + mer_acc[...])[0:1, :]


def _featlogit(h_in, W, al_mat, ar_mat):
    H = al_mat.shape[1]
    return pl.pallas_call(
        _featlogit_body,
        grid=(N // BLK,),
        in_specs=[
            pl.BlockSpec((BLK, HID), lambda i: (i, 0)),
            pl.BlockSpec((HID, HID), lambda i: (0, 0)),
            pl.BlockSpec((HID, H), lambda i: (0, 0)),
            pl.BlockSpec((HID, H), lambda i: (0, 0)),
        ],
        out_specs=[
            pl.BlockSpec((BLK, HID), lambda i: (i, 0)),
            pl.BlockSpec((BLK, H), lambda i: (i, 0)),
            pl.BlockSpec((BLK, H), lambda i: (i, 0)),
            pl.BlockSpec((1, 128), lambda i: (0, 0)),
        ],
        out_shape=[
            jax.ShapeDtypeStruct((N, HID), jnp.float32),
            jax.ShapeDtypeStruct((N, H), jnp.float32),
            jax.ShapeDtypeStruct((N, H), jnp.float32),
            jax.ShapeDtypeStruct((1, 128), jnp.float32),
        ],
        scratch_shapes=[
            pltpu.VMEM((8, 128), jnp.float32),
            pltpu.VMEM((8, 128), jnp.float32),
        ],
    )(h_in, W, al_mat, ar_mat)


def _gat2feat_body(rstp_ref, denp_ref, b1_ref, w2_ref, alm_ref, arm_ref,
                   feat2_ref, el2_ref, er2_ref, m_ref, mel_acc, mer_acc):
    i = pl.program_id(0)
    rr = rstp_ref[...]
    r = rr[0] + rr[1]
    dd = denp_ref[...]
    dinv = 1.0 / (dd[0] + dd[1] + 1e-9)
    colh = lax.broadcasted_iota(jnp.int32, (1, 128), 1) // 64
    acc = jnp.zeros((rstp_ref.shape[2], HID), jnp.float32)
    for cc in range(4):
        d0 = dinv[:, 2 * cc:2 * cc + 1]
        d1 = dinv[:, 2 * cc + 1:2 * cc + 2]
        scale = jnp.where(colh == 0, d0, d1)
        h = _elu(r[cc] * scale + b1_ref[...][cc][None, :])
        acc += jnp.dot(h, w2_ref[...][cc], preferred_element_type=jnp.float32)
    feat2_ref[...] = acc
    el = jnp.dot(acc, alm_ref[...], preferred_element_type=jnp.float32)
    er = jnp.dot(acc, arm_ref[...], preferred_element_type=jnp.float32)
    el2_ref[...] = el
    er2_ref[...] = er

    @pl.when(i == 0)
    def _():
        mel_acc[...] = jnp.full((8, 128), NEG, jnp.float32)
        mer_acc[...] = jnp.full((8, 128), NEG, jnp.float32)

    mel_acc[...] = jnp.maximum(mel_acc[...], jnp.full((8, 128), jnp.max(el)))
    mer_acc[...] = jnp.maximum(mer_acc[...], jnp.full((8, 128), jnp.max(er)))
    m_ref[...] = (mel_acc[...] + mer_acc[...])[0:1, :]


def _gat2feat(rstp, denp, b1c, w2c, al2, ar2):
    return pl.pallas_call(
        _gat2feat_body,
        grid=(N // BLK,),
        in_specs=[
            pl.BlockSpec((2, 4, BLK, 128), lambda i: (0, 0, i, 0)),
            pl.BlockSpec((2, BLK, 8), lambda i: (0, i, 0)),
            pl.BlockSpec((4, 128), lambda i: (0, 0)),
            pl.BlockSpec((4, 128, HID), lambda i: (0, 0, 0)),
            pl.BlockSpec((HID, 1), lambda i: (0, 0)),
            pl.BlockSpec((HID, 1), lambda i: (0, 0)),
        ],
        out_specs=[
            pl.BlockSpec((BLK, HID), lambda i: (i, 0)),
            pl.BlockSpec((BLK, 1), lambda i: (i, 0)),
            pl.BlockSpec((BLK, 1), lambda i: (i, 0)),
            pl.BlockSpec((1, 128), lambda i: (0, 0)),
        ],
        out_shape=[
            jax.ShapeDtypeStruct((N, HID), jnp.float32),
            jax.ShapeDtypeStruct((N, 1), jnp.float32),
            jax.ShapeDtypeStruct((N, 1), jnp.float32),
            jax.ShapeDtypeStruct((1, 128), jnp.float32),
        ],
        scratch_shapes=[
            pltpu.VMEM((8, 128), jnp.float32),
            pltpu.VMEM((8, 128), jnp.float32),
        ],
    )(rstp, denp, b1c, w2c, al2, ar2)


def _final_body(r0_ref, d0_ref, b20_ref, r1_ref, d1_ref, b21_ref,
                attn_ref, clsw_ref, clsb_ref, out_ref):
    blk = out_ref.shape[0]

    def branch_h(r_ref, d_ref, b2_ref):
        rr = r_ref[...]
        r = rr[0] + rr[1]
        dd = d_ref[...]
        dinv = 1.0 / (dd[0] + dd[1] + 1e-9)
        hs = []
        s = jnp.zeros((blk, 1), jnp.float32)
        for cc in range(4):
            h = r[cc] * dinv + b2_ref[...][cc][None, :]
            hs.append(h)
            s += jnp.dot(h, attn_ref[...][cc],
                         preferred_element_type=jnp.float32)
        return hs, s

    h0, s0 = branch_h(r0_ref, d0_ref, b20_ref)
    h1, s1 = branch_h(r1_ref, d1_ref, b21_ref)
    m = jnp.maximum(s0, s1)
    e0 = jnp.exp(s0 - m)
    e1 = jnp.exp(s1 - m)
    a0 = e0 / (e0 + e1)
    a1 = 1.0 - a0
    acc = jnp.zeros((blk, OUT), jnp.float32)
    for cc in range(4):
        hc = a0 * h0[cc] + a1 * h1[cc]
        acc += jnp.dot(hc, clsw_ref[...][cc],
                       preferred_element_type=jnp.float32)
    out_ref[...] = acc + clsb_ref[...]


def _final(r0, d0, b20, r1, d1, b21, attn_c, clsw_c, cls_b):
    blk = 1000
    return pl.pallas_call(
        _final_body,
        grid=(N // blk,),
        in_specs=[
            pl.BlockSpec((2, 4, blk, 128), lambda i: (0, 0, i, 0)),
            pl.BlockSpec((2, blk, 1), lambda i: (0, i, 0)),
            pl.BlockSpec((4, 128), lambda i: (0, 0)),
            pl.BlockSpec((2, 4, blk, 128), lambda i: (0, 0, i, 0)),
            pl.BlockSpec((2, blk, 1), lambda i: (0, i, 0)),
            pl.BlockSpec((4, 128), lambda i: (0, 0)),
            pl.BlockSpec((4, 128, 1), lambda i: (0, 0, 0)),
            pl.BlockSpec((4, 128, OUT), lambda i: (0, 0, 0)),
            pl.BlockSpec((1, OUT), lambda i: (0, 0)),
        ],
        out_specs=pl.BlockSpec((blk, OUT), lambda i: (i, 0)),
        out_shape=jax.ShapeDtypeStruct((N, OUT), jnp.float32),
    )(r0, d0, b20, r1, d1, b21, attn_c, clsw_c, cls_b.reshape(1, OUT))


NC = 2
NS = 16
NW = NC * NS
EPW = E // NW
NPT = N // NS
BA = 1000
BB = 200


def _sc_edge_attn(src, dst, elf, erf, M, H):
    mesh = plsc.VectorSubcoreMesh(core_axis_name="c", subcore_axis_name="s")
    FS = H * BA
    NGRP = (FS + 15) // 16
    NIDX = (BA + 15) // 16
    ZB = 8 * ((N * H) // (NS * 8))
    ZT = N * H - NS * ZB

    @functools.partial(
        pl.kernel,
        out_type=[jax.ShapeDtypeStruct((E,), jnp.float32) for _ in range(H)]
        + [jax.ShapeDtypeStruct((NC * N * H,), jnp.float32)],
        name="sc_edge_attn",
        mesh=mesh,
        scratch_types=[
            pltpu.VMEM((BA,), jnp.int32),
            pltpu.VMEM((BA,), jnp.int32),
            pltpu.VMEM((FS,), jnp.int32),
            pltpu.VMEM((FS,), jnp.int32),
            pltpu.VMEM((FS,), jnp.float32),
            pltpu.VMEM((FS,), jnp.float32),
            pltpu.VMEM((FS,), jnp.float32),
            pltpu.VMEM((128,), jnp.float32),
            pltpu.VMEM_SHARED((N * H,), jnp.float32),
            pltpu.SemaphoreType.DMA,
        ],
    )
    def k(src_hbm, dst_hbm, el_hbm, er_hbm, m_hbm, *rest):
        eets = rest[:H]
        den_hbm = rest[H]
        (srcv, dstv, idxs, idxd, ela, era, eea, mv, den_sh, sem) = rest[H + 1:]
        c = lax.axis_index("c")
        s = lax.axis_index("s")
        wid = s * NC + c
        pltpu.sync_copy(m_hbm.at[0], mv)
        mvec = mv[pl.ds(0, 16)]

        def zg(g, _):
            o = jnp.minimum(g * 16, FS - 16)
            eea[pl.ds(o, 16)] = jnp.zeros((16,), jnp.float32)
            return 0

        lax.fori_loop(0, NGRP, zg, 0)
        pltpu.sync_copy(eea.at[pl.ds(0, ZB)], den_sh.at[pl.ds(s * ZB, ZB)])
        if ZT:
            @pl.when(s == NS - 1)
            def _():
                pltpu.sync_copy(eea.at[pl.ds(0, ZT)],
                                den_sh.at[pl.ds(NS * ZB, ZT)])
        plsc.subcore_barrier()

        def chunk(kk, _):
            base = wid * EPW + kk * BA
            pltpu.sync_copy(src_hbm.at[pl.ds(base, BA)], srcv)
            pltpu.sync_copy(dst_hbm.at[pl.ds(base, BA)], dstv)
            for h in range(H):
                def bld(g, _):
                    o = jnp.minimum(g * 16, BA - 16)
                    sv = srcv[pl.ds(o, 16)]
                    dv = dstv[pl.ds(o, 16)]
                    idxs[pl.ds(h * BA + o, 16)] = sv * H + h
                    idxd[pl.ds(h * BA + o, 16)] = dv * H + h
                    return 0

                lax.fori_loop(0, NIDX, bld, 0)
            cp1 = pltpu.async_copy(el_hbm.at[idxs], ela, sem)
            cp2 = pltpu.async_copy(er_hbm.at[idxd], era, sem)
            cp1.wait()
            cp2.wait()

            def grp(g, _):
                o = jnp.minimum(g * 16, FS - 16)
                ssum = ela[pl.ds(o, 16)] + era[pl.ds(o, 16)]
                e = jnp.maximum(ssum, ssum * jnp.float32(0.2))
                eea[pl.ds(o, 16)] = jnp.exp(e - mvec)
                return 0

            lax.fori_loop(0, NGRP, grp, 0)
            pltpu.sync_copy(eea, den_sh.at[idxd], add=True)
            for h in range(H):
                pltpu.sync_copy(eea.at[pl.ds(h * BA, BA)],
                                eets[h].at[pl.ds(base, BA)])
            return 0

        lax.fori_loop(0, EPW // BA, chunk, 0)
        plsc.subcore_barrier()
        pltpu.sync_copy(den_sh.at[pl.ds(s * ZB, ZB)], eea.at[pl.ds(0, ZB)])
        pltpu.sync_copy(eea.at[pl.ds(0, ZB)],
                        den_hbm.at[pl.ds(c * N * H + s * ZB, ZB)])
        if ZT:
            @pl.when(s == NS - 1)
            def _():
                pltpu.sync_copy(den_sh.at[pl.ds(NS * ZB, ZT)],
                                eea.at[pl.ds(0, ZT)])
                pltpu.sync_copy(eea.at[pl.ds(0, ZT)],
                                den_hbm.at[pl.ds(c * N * H + NS * ZB, ZT)])

    outs = k(src, dst, elf, erf, M)
    return outs[:H], outs[H].reshape(NC, N, H)


def _sc_aggregate(src, dst, eets, fcs, H):
    mesh = plsc.VectorSubcoreMesh(core_axis_name="c", subcore_axis_name="s")
    ZB = 8 * (N // (NS * 8))
    ZT = N - NS * ZB

    @functools.partial(
        pl.kernel,
        out_type=jax.ShapeDtypeStruct((NC, 4, N, 128), jnp.float32),
        name="sc_aggregate",
        mesh=mesh,
        scratch_types=[
            pltpu.VMEM((BB,), jnp.int32),
            pltpu.VMEM((BB,), jnp.int32),
            pltpu.VMEM((BB,), jnp.float32),
            pltpu.VMEM((BB,), jnp.float32),
            pltpu.VMEM((BB, 128), jnp.float32),
            pltpu.VMEM((BB * 16,), jnp.int32),
            pltpu.VMEM((BB * 16,), jnp.int32),
            pltpu.VMEM((BB * 16,), jnp.float32),
            pltpu.VMEM((BB * 16,), jnp.float32),
            pltpu.VMEM_SHARED((NS * 2 * BB,), jnp.float32),
            pltpu.VMEM_SHARED((N, 128), jnp.float32),
            pltpu.SemaphoreType.DMA,
        ],
    )
    def k(src_hbm, dst_hbm, f0, f1, f2, f3, *rest):
        eeh = rest[:H]
        out_hbm = rest[H]
        (srcv, dstv, w0v, w1v, rows, idxw0, idxw1, w0x, w1x,
         wsp, rst_sh, sem) = rest[H + 1:]
        c = lax.axis_index("c")
        s = lax.axis_index("s")
        wid = s * NC + c
        fr = [f0, f1, f2, f3]

        def mkrep(e, _):
            idxw0[pl.ds(e * 16, 16)] = jnp.full((16,), s * 2 * BB + e,
                                                jnp.int32)
            idxw1[pl.ds(e * 16, 16)] = jnp.full((16,), s * 2 * BB + BB + e,
                                                jnp.int32)
            return 0

        lax.fori_loop(0, BB, mkrep, 0)
        for cc in range(4):
            h0 = min(2 * cc, H - 1)
            h1 = min(2 * cc + 1, H - 1)
            def zr(e, _):
                for j in range(8):
                    rows[e, pl.ds(j * 16, 16)] = jnp.zeros((16,), jnp.float32)
                return 0

            lax.fori_loop(0, BB, zr, 0)
            for i in range(3):
                pltpu.sync_copy(
                    rows, rst_sh.at[pl.ds(s * ZB + i * BB, BB)])
            pltpu.sync_copy(rows.at[pl.ds(0, 24)],
                            rst_sh.at[pl.ds(s * ZB + 3 * BB, 24)])

            @pl.when(s == NS - 1)
            def _():
                pltpu.sync_copy(rows.at[pl.ds(0, ZT)],
                                rst_sh.at[pl.ds(NS * ZB, ZT)])

            plsc.subcore_barrier()

            def chunk(kk, _):
                base = wid * EPW + kk * BB
                pltpu.sync_copy(src_hbm.at[pl.ds(base, BB)], srcv)
                pltpu.sync_copy(dst_hbm.at[pl.ds(base, BB)], dstv)
                pltpu.sync_copy(eeh[h0].at[pl.ds(base, BB)], w0v)
                pltpu.sync_copy(eeh[h1].at[pl.ds(base, BB)], w1v)
                pltpu.sync_copy(w0v, wsp.at[pl.ds(s * 2 * BB, BB)])
                pltpu.sync_copy(w1v, wsp.at[pl.ds(s * 2 * BB + BB, BB)])
                pltpu.async_copy(wsp.at[idxw0], w0x, sem).wait()
                pltpu.async_copy(wsp.at[idxw1], w1x, sem).wait()
                pltpu.async_copy(fr[cc].at[srcv], rows, sem).wait()

                def edge(e, _):
                    wa = w0x[pl.ds(e * 16, 16)]
                    wb = w1x[pl.ds(e * 16, 16)]
                    for j in range(4):
                        rows[e, pl.ds(j * 16, 16)] = (
                            rows[e, pl.ds(j * 16, 16)] * wa)
                    for j in range(4, 8):
                        rows[e, pl.ds(j * 16, 16)] = (
                            rows[e, pl.ds(j * 16, 16)] * wb)
                    return 0

                lax.fori_loop(0, BB, edge, 0)
                pltpu.sync_copy(rows, rst_sh.at[dstv], add=True)
                return 0

            lax.fori_loop(0, EPW // BB, chunk, 0)
            plsc.subcore_barrier()
            for i in range(3):
                pltpu.sync_copy(rst_sh.at[pl.ds(s * ZB + i * BB, BB)], rows)
                pltpu.sync_copy(
                    rows, out_hbm.at[c, cc, pl.ds(s * ZB + i * BB, BB)])
            pltpu.sync_copy(rst_sh.at[pl.ds(s * ZB + 3 * BB, 24)],
                            rows.at[pl.ds(0, 24)])
            pltpu.sync_copy(rows.at[pl.ds(0, 24)],
                            out_hbm.at[c, cc, pl.ds(s * ZB + 3 * BB, 24)])

            @pl.when(s == NS - 1)
            def _():
                pltpu.sync_copy(rst_sh.at[pl.ds(NS * ZB, ZT)],
                                rows.at[pl.ds(0, ZT)])
                pltpu.sync_copy(rows.at[pl.ds(0, ZT)],
                                out_hbm.at[c, cc, pl.ds(NS * ZB, ZT)])

            plsc.subcore_barrier()

    return k(src, dst, fcs[0], fcs[1], fcs[2], fcs[3], *eets)


def _edge_phase_sc(el, er, M, src, dst, feat, H):
    eets, denp = _sc_edge_attn(src, dst, el.reshape(-1), er.reshape(-1), M, H)
    fcs = [feat[:, cc * 128:(cc + 1) * 128] for cc in range(4)]
    rstp = _sc_aggregate(src, dst, eets, fcs, H)
    return rstp, denp


def _edge_phase_jnp(el, er, M, src, dst, feat, H):
    mscal = M[0, 0]
    e = jax.nn.leaky_relu(el[src] + er[dst], 0.2)
    ee = jnp.exp(e - mscal)
    den = jax.ops.segment_sum(ee, dst, num_segments=N)
    oph = HID // H
    feath = feat.reshape(N, H, oph)
    msg = feath[src] * ee[:, :, None]
    rst = jax.ops.segment_sum(msg, dst, num_segments=N)
    rstp = jnp.stack([rst.reshape(N, 4, 128).transpose(1, 0, 2),
                      jnp.zeros((4, N, 128), jnp.float32)])
    denp = jnp.stack([den, jnp.zeros((N, H), jnp.float32)])
    return rstp, denp


def _head_mat(a):
    H, oph = a.shape
    eye = jnp.eye(H, dtype=a.dtype)
    return (a[:, :, None] * eye[:, None, :]).reshape(H * oph, H)


def kernel(x, edge_index_0, edge_index_1, proj_W, proj_b,
           gat1_W_0, gat1_al_0, gat1_ar_0, gat1_b_0,
           gat2_W_0, gat2_al_0, gat2_ar_0, gat2_b_0,
           gat1_W_1, gat1_al_1, gat1_ar_1, gat1_b_1,
           gat2_W_1, gat2_al_1, gat2_ar_1, gat2_b_1,
           attn_w, cls_W, cls_b):
    h_proj = _proj(x, proj_W, proj_b)
    branches = [
        (edge_index_0, gat1_W_0, gat1_al_0, gat1_ar_0, gat1_b_0,
         gat2_W_0, gat2_al_0, gat2_ar_0, gat2_b_0),
        (edge_index_1, gat1_W_1, gat1_al_1, gat1_ar_1, gat1_b_1,
         gat2_W_1, gat2_al_1, gat2_ar_1, gat2_b_1),
    ]
    outs = []
    for (ei, g1W, g1al, g1ar, g1b, g2W, g2al, g2ar, g2b) in branches:
        src = ei[0]
        dst = ei[1]
        feat1, el1, er1, M1 = _featlogit(h_proj, g1W,
                                         _head_mat(g1al), _head_mat(g1ar))
        rstp1, denp1 = _edge_phase_sc(el1, er1, M1, src, dst, feat1, HEADS)
        feat2, el2, er2, M2 = _gat2feat(
            rstp1, denp1, g1b.reshape(4, 128), g2W.reshape(4, 128, HID),
            g2al.reshape(HID, 1), g2ar.reshape(HID, 1))
        rstp2, denp2 = _edge_phase_sc(el2, er2, M2, src, dst, feat2, 1)
        outs.append((rstp2, denp2, g2b.reshape(4, 128)))
    (r0, d0, b20), (r1, d1, b21) = outs
    return _final(r0, d0, b20, r1, d1, b21,
                  attn_w.reshape(4, 128, 1), cls_W.reshape(4, 128, OUT), cls_b)


# speedup vs baseline: 12.3079x; 1.0710x over previous
import functools

import jax
import jax.numpy as jnp
from jax import lax
from jax.experimental import pallas as pl
from jax.experimental.pallas import tpu as pltpu
from jax.experimental.pallas import tpu_sc as plsc

N = 10000
E = 160000
IN_DIM = 256
HID = 512
HEADS = 8
HPH = 64
OUT = 64

BLK = 1000
NEG = -1e30


def _elu(v):
    return jnp.where(v > 0, v, jnp.exp(jnp.minimum(v, 0.0)) - 1.0)


def _proj_body(x_ref, w_ref, b_ref, o_ref):
    acc = jnp.dot(x_ref[...], w_ref[...], preferred_element_type=jnp.float32)
    o_ref[...] = _elu(acc + b_ref[...])


def _proj(x, w, b):
    return pl.pallas_call(
        _proj_body,
        grid=(N // BLK,),
        in_specs=[
            pl.BlockSpec((BLK, IN_DIM), lambda i: (i, 0)),
            pl.BlockSpec((IN_DIM, HID), lambda i: (0, 0)),
            pl.BlockSpec((1, HID), lambda i: (0, 0)),
        ],
        out_specs=pl.BlockSpec((BLK, HID), lambda i: (i, 0)),
        out_shape=jax.ShapeDtypeStruct((N, HID), jnp.float32),
    )(x, w, b.reshape(1, HID))


def _featlogit_body(h_ref, w_ref, alm_ref, arm_ref,
                    feat_ref, el_ref, er_ref, m_ref, mel_acc, mer_acc):
    i = pl.program_id(0)
    f = jnp.dot(h_ref[...], w_ref[...], preferred_element_type=jnp.float32)
    feat_ref[...] = f
    el = jnp.dot(f, alm_ref[...], preferred_element_type=jnp.float32)
    er = jnp.dot(f, arm_ref[...], preferred_element_type=jnp.float32)
    el_ref[...] = el
    er_ref[...] = er

    @pl.when(i == 0)
    def _():
        mel_acc[...] = jnp.full((8, 128), NEG, jnp.float32)
        mer_acc[...] = jnp.full((8, 128), NEG, jnp.float32)

    mel_acc[...] = jnp.maximum(mel_acc[...], jnp.full((8, 128), jnp.max(el)))
    mer_acc[...] = jnp.maximum(mer_acc[...], jnp.full((8, 128), jnp.max(er)))
    m_ref[...] = (mel_acc[...] + mer_acc[...])[0:1, :]


def _featlogit(h_in, W, al_mat, ar_mat):
    H = al_mat.shape[1]
    return pl.pallas_call(
        _featlogit_body,
        grid=(N // BLK,),
        in_specs=[
            pl.BlockSpec((BLK, HID), lambda i: (i, 0)),
            pl.BlockSpec((HID, HID), lambda i: (0, 0)),
            pl.BlockSpec((HID, H), lambda i: (0, 0)),
            pl.BlockSpec((HID, H), lambda i: (0, 0)),
        ],
        out_specs=[
            pl.BlockSpec((BLK, HID), lambda i: (i, 0)),
            pl.BlockSpec((BLK, H), lambda i: (i, 0)),
            pl.BlockSpec((BLK, H), lambda i: (i, 0)),
            pl.BlockSpec((1, 128), lambda i: (0, 0)),
        ],
        out_shape=[
            jax.ShapeDtypeStruct((N, HID), jnp.float32),
            jax.ShapeDtypeStruct((N, H), jnp.float32),
            jax.ShapeDtypeStruct((N, H), jnp.float32),
            jax.ShapeDtypeStruct((1, 128), jnp.float32),
        ],
        scratch_shapes=[
            pltpu.VMEM((8, 128), jnp.float32),
            pltpu.VMEM((8, 128), jnp.float32),
        ],
    )(h_in, W, al_mat, ar_mat)


def _gat2feat_body(rstp_ref, denp_ref, b1_ref, w2_ref, alm_ref, arm_ref,
                   feat2_ref, el2_ref, er2_ref, m_ref, mel_acc, mer_acc):
    i = pl.program_id(0)
    rr = rstp_ref[...]
    r = rr[0] + rr[1]
    dd = denp_ref[...]
    dinv = 1.0 / (dd[0] + dd[1] + 1e-9)
    colh = lax.broadcasted_iota(jnp.int32, (1, 128), 1) // 64
    acc = jnp.zeros((rstp_ref.shape[2], HID), jnp.float32)
    for cc in range(4):
        d0 = dinv[:, 2 * cc:2 * cc + 1]
        d1 = dinv[:, 2 * cc + 1:2 * cc + 2]
        scale = jnp.where(colh == 0, d0, d1)
        h = _elu(r[cc] * scale + b1_ref[...][cc][None, :])
        acc += jnp.dot(h, w2_ref[...][cc], preferred_element_type=jnp.float32)
    feat2_ref[...] = acc
    el = jnp.dot(acc, alm_ref[...], preferred_element_type=jnp.float32)
    er = jnp.dot(acc, arm_ref[...], preferred_element_type=jnp.float32)
    el2_ref[...] = el
    er2_ref[...] = er

    @pl.when(i == 0)
    def _():
        mel_acc[...] = jnp.full((8, 128), NEG, jnp.float32)
        mer_acc[...] = jnp.full((8, 128), NEG, jnp.float32)

    mel_acc[...] = jnp.maximum(mel_acc[...], jnp.full((8, 128), jnp.max(el)))
    mer_acc[...] = jnp.maximum(mer_acc[...], jnp.full((8, 128), jnp.max(er)))
    m_ref[...] = (mel_acc[...] + mer_acc[...])[0:1, :]


def _gat2feat(rstp, denp, b1c, w2c, al2, ar2):
    return pl.pallas_call(
        _gat2feat_body,
        grid=(N // BLK,),
        in_specs=[
            pl.BlockSpec((2, 4, BLK, 128), lambda i: (0, 0, i, 0)),
            pl.BlockSpec((2, BLK, 8), lambda i: (0, i, 0)),
            pl.BlockSpec((4, 128), lambda i: (0, 0)),
            pl.BlockSpec((4, 128, HID), lambda i: (0, 0, 0)),
            pl.BlockSpec((HID, 1), lambda i: (0, 0)),
            pl.BlockSpec((HID, 1), lambda i: (0, 0)),
        ],
        out_specs=[
            pl.BlockSpec((BLK, HID), lambda i: (i, 0)),
            pl.BlockSpec((BLK, 1), lambda i: (i, 0)),
            pl.BlockSpec((BLK, 1), lambda i: (i, 0)),
            pl.BlockSpec((1, 128), lambda i: (0, 0)),
        ],
        out_shape=[
            jax.ShapeDtypeStruct((N, HID), jnp.float32),
            jax.ShapeDtypeStruct((N, 1), jnp.float32),
            jax.ShapeDtypeStruct((N, 1), jnp.float32),
            jax.ShapeDtypeStruct((1, 128), jnp.float32),
        ],
        scratch_shapes=[
            pltpu.VMEM((8, 128), jnp.float32),
            pltpu.VMEM((8, 128), jnp.float32),
        ],
    )(rstp, denp, b1c, w2c, al2, ar2)


def _final_body(r0_ref, d0_ref, b20_ref, r1_ref, d1_ref, b21_ref,
                attn_ref, clsw_ref, clsb_ref, out_ref):
    blk = out_ref.shape[0]

    def branch_h(r_ref, d_ref, b2_ref):
        rr = r_ref[...]
        r = rr[0] + rr[1]
        dd = d_ref[...]
        dinv = 1.0 / (dd[0] + dd[1] + 1e-9)
        hs = []
        s = jnp.zeros((blk, 1), jnp.float32)
        for cc in range(4):
            h = r[cc] * dinv + b2_ref[...][cc][None, :]
            hs.append(h)
            s += jnp.dot(h, attn_ref[...][cc],
                         preferred_element_type=jnp.float32)
        return hs, s

    h0, s0 = branch_h(r0_ref, d0_ref, b20_ref)
    h1, s1 = branch_h(r1_ref, d1_ref, b21_ref)
    m = jnp.maximum(s0, s1)
    e0 = jnp.exp(s0 - m)
    e1 = jnp.exp(s1 - m)
    a0 = e0 / (e0 + e1)
    a1 = 1.0 - a0
    acc = jnp.zeros((blk, OUT), jnp.float32)
    for cc in range(4):
        hc = a0 * h0[cc] + a1 * h1[cc]
        acc += jnp.dot(hc, clsw_ref[...][cc],
                       preferred_element_type=jnp.float32)
    out_ref[...] = acc + clsb_ref[...]


def _final(r0, d0, b20, r1, d1, b21, attn_c, clsw_c, cls_b):
    blk = 1000
    return pl.pallas_call(
        _final_body,
        grid=(N // blk,),
        in_specs=[
            pl.BlockSpec((2, 4, blk, 128), lambda i: (0, 0, i, 0)),
            pl.BlockSpec((2, blk, 1), lambda i: (0, i, 0)),
            pl.BlockSpec((4, 128), lambda i: (0, 0)),
            pl.BlockSpec((2, 4, blk, 128), lambda i: (0, 0, i, 0)),
            pl.BlockSpec((2, blk, 1), lambda i: (0, i, 0)),
            pl.BlockSpec((4, 128), lambda i: (0, 0)),
            pl.BlockSpec((4, 128, 1), lambda i: (0, 0, 0)),
            pl.BlockSpec((4, 128, OUT), lambda i: (0, 0, 0)),
            pl.BlockSpec((1, OUT), lambda i: (0, 0)),
        ],
        out_specs=pl.BlockSpec((blk, OUT), lambda i: (i, 0)),
        out_shape=jax.ShapeDtypeStruct((N, OUT), jnp.float32),
    )(r0, d0, b20, r1, d1, b21, attn_c, clsw_c, cls_b.reshape(1, OUT))


NC = 2
NS = 16
NW = NC * NS
EPW = E // NW
NPT = N // NS
BA = 1000
BB = 200


def _sc_edge_attn(src, dst, elf, erf, M, H):
    mesh = plsc.VectorSubcoreMesh(core_axis_name="c", subcore_axis_name="s")
    FS = H * BA
    NGRP = (FS + 15) // 16
    NIDX = (BA + 15) // 16
    ZB = 8 * ((N * H) // (NS * 8))
    ZT = N * H - NS * ZB

    @functools.partial(
        pl.kernel,
        out_type=[jax.ShapeDtypeStruct((E,), jnp.float32) for _ in range(H)]
        + [jax.ShapeDtypeStruct((NC * N * H,), jnp.float32)],
        name="sc_edge_attn",
        mesh=mesh,
        scratch_types=[
            pltpu.VMEM((BA,), jnp.int32),
            pltpu.VMEM((BA,), jnp.int32),
            pltpu.VMEM((FS,), jnp.int32),
            pltpu.VMEM((FS,), jnp.int32),
            pltpu.VMEM((FS,), jnp.float32),
            pltpu.VMEM((FS,), jnp.float32),
            pltpu.VMEM((FS,), jnp.float32),
            pltpu.VMEM((128,), jnp.float32),
            pltpu.VMEM_SHARED((N * H,), jnp.float32),
            pltpu.SemaphoreType.DMA,
        ],
    )
    def k(src_hbm, dst_hbm, el_hbm, er_hbm, m_hbm, *rest):
        eets = rest[:H]
        den_hbm = rest[H]
        (srcv, dstv, idxs, idxd, ela, era, eea, mv, den_sh, sem) = rest[H + 1:]
        c = lax.axis_index("c")
        s = lax.axis_index("s")
        wid = s * NC + c
        pltpu.sync_copy(m_hbm.at[0], mv)
        mvec = mv[pl.ds(0, 16)]

        def zg(g, _):
            o = jnp.minimum(g * 16, FS - 16)
            eea[pl.ds(o, 16)] = jnp.zeros((16,), jnp.float32)
            return 0

        lax.fori_loop(0, NGRP, zg, 0)
        pltpu.sync_copy(eea.at[pl.ds(0, ZB)], den_sh.at[pl.ds(s * ZB, ZB)])
        if ZT:
            @pl.when(s == NS - 1)
            def _():
                pltpu.sync_copy(eea.at[pl.ds(0, ZT)],
                                den_sh.at[pl.ds(NS * ZB, ZT)])
        plsc.subcore_barrier()

        def chunk(kk, _):
            base = wid * EPW + kk * BA
            pltpu.sync_copy(src_hbm.at[pl.ds(base, BA)], srcv)
            pltpu.sync_copy(dst_hbm.at[pl.ds(base, BA)], dstv)
            for h in range(H):
                def bld(g, _):
                    o = jnp.minimum(g * 16, BA - 16)
                    sv = srcv[pl.ds(o, 16)]
                    dv = dstv[pl.ds(o, 16)]
                    idxs[pl.ds(h * BA + o, 16)] = sv * H + h
                    idxd[pl.ds(h * BA + o, 16)] = dv * H + h
                    return 0

                lax.fori_loop(0, NIDX, bld, 0)
            cp1 = pltpu.async_copy(el_hbm.at[idxs], ela, sem)
            cp2 = pltpu.async_copy(er_hbm.at[idxd], era, sem)
            cp1.wait()
            cp2.wait()

            def grp(g, _):
                o = jnp.minimum(g * 16, FS - 16)
                ssum = ela[pl.ds(o, 16)] + era[pl.ds(o, 16)]
                e = jnp.maximum(ssum, ssum * jnp.float32(0.2))
                eea[pl.ds(o, 16)] = jnp.exp(e - mvec)
                return 0

            lax.fori_loop(0, NGRP, grp, 0)
            pltpu.sync_copy(eea, den_sh.at[idxd], add=True)
            for h in range(H):
                pltpu.sync_copy(eea.at[pl.ds(h * BA, BA)],
                                eets[h].at[pl.ds(base, BA)])
            return 0

        lax.fori_loop(0, EPW // BA, chunk, 0)
        plsc.subcore_barrier()
        pltpu.sync_copy(den_sh.at[pl.ds(s * ZB, ZB)], eea.at[pl.ds(0, ZB)])
        pltpu.sync_copy(eea.at[pl.ds(0, ZB)],
                        den_hbm.at[pl.ds(c * N * H + s * ZB, ZB)])
        if ZT:
            @pl.when(s == NS - 1)
            def _():
                pltpu.sync_copy(den_sh.at[pl.ds(NS * ZB, ZT)],
                                eea.at[pl.ds(0, ZT)])
                pltpu.sync_copy(eea.at[pl.ds(0, ZT)],
                                den_hbm.at[pl.ds(c * N * H + NS * ZB, ZT)])

    outs = k(src, dst, elf, erf, M)
    return outs[:H], outs[H].reshape(NC, N, H)


def _sc_aggregate(src, dst, eets, fcs, H):
    mesh = plsc.VectorSubcoreMesh(core_axis_name="c", subcore_axis_name="s")
    ZB = 8 * (N // (NS * 8))
    ZT = N - NS * ZB

    @functools.partial(
        pl.kernel,
        out_type=jax.ShapeDtypeStruct((NC, 4, N, 128), jnp.float32),
        name="sc_aggregate",
        mesh=mesh,
        scratch_types=[
            pltpu.VMEM((BB,), jnp.int32),
            pltpu.VMEM((BB,), jnp.int32),
            pltpu.VMEM((BB,), jnp.float32),
            pltpu.VMEM((BB,), jnp.float32),
            pltpu.VMEM((BB, 128), jnp.float32),
            pltpu.VMEM((BB * 16,), jnp.int32),
            pltpu.VMEM((BB * 16,), jnp.int32),
            pltpu.VMEM((BB * 16,), jnp.float32),
            pltpu.VMEM((BB * 16,), jnp.float32),
            pltpu.VMEM_SHARED((NS * 2 * BB,), jnp.float32),
            pltpu.VMEM_SHARED((N, 128), jnp.float32),
            pltpu.SemaphoreType.DMA,
        ],
    )
    def k(src_hbm, dst_hbm, f0, f1, f2, f3, *rest):
        eeh = rest[:H]
        out_hbm = rest[H]
        (srcv, dstv, w0v, w1v, rows, idxw0, idxw1, w0x, w1x,
         wsp, rst_sh, sem) = rest[H + 1:]
        c = lax.axis_index("c")
        s = lax.axis_index("s")
        wid = s * NC + c
        fr = [f0, f1, f2, f3]

        def mkrep(e, _):
            idxw0[pl.ds(e * 16, 16)] = jnp.full((16,), s * 2 * BB + e,
                                                jnp.int32)
            idxw1[pl.ds(e * 16, 16)] = jnp.full((16,), s * 2 * BB + BB + e,
                                                jnp.int32)
            return 0

        lax.fori_loop(0, BB, mkrep, 0)
        for cc in range(4):
            h0 = min(2 * cc, H - 1)
            h1 = min(2 * cc + 1, H - 1)
            def zr(e, _):
                for j in range(8):
                    rows[e, pl.ds(j * 16, 16)] = jnp.zeros((16,), jnp.float32)
                return 0

            lax.fori_loop(0, BB, zr, 0)
            for i in range(3):
                pltpu.sync_copy(
                    rows, rst_sh.at[pl.ds(s * ZB + i * BB, BB)])
            pltpu.sync_copy(rows.at[pl.ds(0, 24)],
                            rst_sh.at[pl.ds(s * ZB + 3 * BB, 24)])

            @pl.when(s == NS - 1)
            def _():
                pltpu.sync_copy(rows.at[pl.ds(0, ZT)],
                                rst_sh.at[pl.ds(NS * ZB, ZT)])

            plsc.subcore_barrier()

            def chunk(kk, _):
                base = wid * EPW + kk * BB
                pltpu.sync_copy(src_hbm.at[pl.ds(base, BB)], srcv)
                pltpu.sync_copy(dst_hbm.at[pl.ds(base, BB)], dstv)
                pltpu.sync_copy(eeh[h0].at[pl.ds(base, BB)], w0v)
                pltpu.sync_copy(eeh[h1].at[pl.ds(base, BB)], w1v)
                pltpu.sync_copy(w0v, wsp.at[pl.ds(s * 2 * BB, BB)])
                pltpu.sync_copy(w1v, wsp.at[pl.ds(s * 2 * BB + BB, BB)])
                pltpu.async_copy(wsp.at[idxw0], w0x, sem).wait()
                pltpu.async_copy(wsp.at[idxw1], w1x, sem).wait()
                pltpu.async_copy(fr[cc].at[srcv], rows, sem).wait()

                @plsc.parallel_loop(0, BB, unroll=4)
                def _(e):
                    wa = w0x[pl.ds(e * 16, 16)]
                    wb = w1x[pl.ds(e * 16, 16)]
                    for j in range(4):
                        rows[e, pl.ds(j * 16, 16)] = (
                            rows[e, pl.ds(j * 16, 16)] * wa)
                    for j in range(4, 8):
                        rows[e, pl.ds(j * 16, 16)] = (
                            rows[e, pl.ds(j * 16, 16)] * wb)
                pltpu.sync_copy(rows, rst_sh.at[dstv], add=True)
                return 0

            lax.fori_loop(0, EPW // BB, chunk, 0)
            plsc.subcore_barrier()
            for i in range(3):
                pltpu.sync_copy(rst_sh.at[pl.ds(s * ZB + i * BB, BB)], rows)
                pltpu.sync_copy(
                    rows, out_hbm.at[c, cc, pl.ds(s * ZB + i * BB, BB)])
            pltpu.sync_copy(rst_sh.at[pl.ds(s * ZB + 3 * BB, 24)],
                            rows.at[pl.ds(0, 24)])
            pltpu.sync_copy(rows.at[pl.ds(0, 24)],
                            out_hbm.at[c, cc, pl.ds(s * ZB + 3 * BB, 24)])

            @pl.when(s == NS - 1)
            def _():
                pltpu.sync_copy(rst_sh.at[pl.ds(NS * ZB, ZT)],
                                rows.at[pl.ds(0, ZT)])
                pltpu.sync_copy(rows.at[pl.ds(0, ZT)],
                                out_hbm.at[c, cc, pl.ds(NS * ZB, ZT)])

            plsc.subcore_barrier()

    return k(src, dst, fcs[0], fcs[1], fcs[2], fcs[3], *eets)


def _edge_phase_sc(el, er, M, src, dst, feat, H):
    eets, denp = _sc_edge_attn(src, dst, el.reshape(-1), er.reshape(-1), M, H)
    fcs = [feat[:, cc * 128:(cc + 1) * 128] for cc in range(4)]
    rstp = _sc_aggregate(src, dst, eets, fcs, H)
    return rstp, denp


def _edge_phase_jnp(el, er, M, src, dst, feat, H):
    mscal = M[0, 0]
    e = jax.nn.leaky_relu(el[src] + er[dst], 0.2)
    ee = jnp.exp(e - mscal)
    den = jax.ops.segment_sum(ee, dst, num_segments=N)
    oph = HID // H
    feath = feat.reshape(N, H, oph)
    msg = feath[src] * ee[:, :, None]
    rst = jax.ops.segment_sum(msg, dst, num_segments=N)
    rstp = jnp.stack([rst.reshape(N, 4, 128).transpose(1, 0, 2),
                      jnp.zeros((4, N, 128), jnp.float32)])
    denp = jnp.stack([den, jnp.zeros((N, H), jnp.float32)])
    return rstp, denp


def _head_mat(a):
    H, oph = a.shape
    eye = jnp.eye(H, dtype=a.dtype)
    return (a[:, :, None] * eye[:, None, :]).reshape(H * oph, H)


def kernel(x, edge_index_0, edge_index_1, proj_W, proj_b,
           gat1_W_0, gat1_al_0, gat1_ar_0, gat1_b_0,
           gat2_W_0, gat2_al_0, gat2_ar_0, gat2_b_0,
           gat1_W_1, gat1_al_1, gat1_ar_1, gat1_b_1,
           gat2_W_1, gat2_al_1, gat2_ar_1, gat2_b_1,
           attn_w, cls_W, cls_b):
    h_proj = _proj(x, proj_W, proj_b)
    branches = [
        (edge_index_0, gat1_W_0, gat1_al_0, gat1_ar_0, gat1_b_0,
         gat2_W_0, gat2_al_0, gat2_ar_0, gat2_b_0),
        (edge_index_1, gat1_W_1, gat1_al_1, gat1_ar_1, gat1_b_1,
         gat2_W_1, gat2_al_1, gat2_ar_1, gat2_b_1),
    ]
    outs = []
    for (ei, g1W, g1al, g1ar, g1b, g2W, g2al, g2ar, g2b) in branches:
        src = ei[0]
        dst = ei[1]
        feat1, el1, er1, M1 = _featlogit(h_proj, g1W,
                                         _head_mat(g1al), _head_mat(g1ar))
        rstp1, denp1 = _edge_phase_sc(el1, er1, M1, src, dst, feat1, HEADS)
        feat2, el2, er2, M2 = _gat2feat(
            rstp1, denp1, g1b.reshape(4, 128), g2W.reshape(4, 128, HID),
            g2al.reshape(HID, 1), g2ar.reshape(HID, 1))
        rstp2, denp2 = _edge_phase_sc(el2, er2, M2, src, dst, feat2, 1)
        outs.append((rstp2, denp2, g2b.reshape(4, 128)))
    (r0, d0, b20), (r1, d1, b21) = outs
    return _final(r0, d0, b20, r1, d1, b21,
                  attn_w.reshape(4, 128, 1), cls_W.reshape(4, 128, OUT), cls_b)
